# Initial kernel scaffold; baseline (speedup 1.0000x reference)
import jax
import jax.numpy as jnp
from jax.experimental import pallas as pl


def kernel(x, edge_index, edge_attr, batch, W1, b1, W2, b2, Wc0, bc0, Wc1, bc1, Wc2, bc2):
    raise NotImplementedError("write your pallas kernel here")


# speedup vs baseline: 7.3300x; 7.3300x over previous
import functools

import jax
import jax.numpy as jnp
from jax import lax
from jax.experimental import pallas as pl
from jax.experimental.pallas import tpu as pltpu
from jax.experimental.pallas import tpu_sc as plsc

NC = 2
NS = 16
LANES = 16
WIN = 80


def _mesh():
  return plsc.VectorSubcoreMesh(
      core_axis_name="c", subcore_axis_name="s", num_cores=NC,
      num_subcores=NS)


def _sc_degree(E, n_pad):
  epw = E // (NC * NS)
  nwin = epw // WIN
  rpt = n_pad // NS

  @functools.partial(
      pl.kernel,
      out_type=jax.ShapeDtypeStruct((NC, n_pad), jnp.float32),
      mesh=_mesh(),
      scratch_types=[
          pltpu.VMEM_SHARED((n_pad,), jnp.float32),
          pltpu.VMEM((rpt,), jnp.float32),
          pltpu.VMEM((WIN,), jnp.int32),
          pltpu.VMEM((WIN,), jnp.float32),
      ],
  )
  def deg_kernel(dst_hbm, ew_hbm, out_hbm, acc, zbuf, didx, ewv):
    cid = lax.axis_index("c")
    sid = lax.axis_index("s")
    wid = sid * NC + cid
    base = wid * epw
    for i in range(rpt // LANES):
      zbuf[pl.ds(i * LANES, LANES)] = jnp.zeros((LANES,), jnp.float32)
    r0 = sid * rpt
    pltpu.sync_copy(zbuf, acc.at[pl.ds(r0, rpt)])
    plsc.subcore_barrier()

    def win(w, carry):
      off = base + w * WIN
      pltpu.sync_copy(dst_hbm.at[pl.ds(off, WIN)], didx)
      pltpu.sync_copy(ew_hbm.at[pl.ds(off, WIN)], ewv)
      pltpu.sync_copy(ewv, acc.at[didx], add=True)
      return carry

    lax.fori_loop(0, nwin, win, 0)
    plsc.subcore_barrier()
    pltpu.sync_copy(acc.at[pl.ds(r0, rpt)], out_hbm.at[cid, pl.ds(r0, rpt)])

  return deg_kernel


def _sc_message(N, E, H):
  epw = E // (NC * NS)
  nwin = epw // WIN
  rpt = (N // (8 * NS)) * 8
  rem = N - rpt * NS

  @functools.partial(
      pl.kernel,
      out_type=jax.ShapeDtypeStruct((NC, N, H), jnp.float32),
      mesh=_mesh(),
      scratch_types=[
          pltpu.VMEM_SHARED((N, H), jnp.float32),
          pltpu.VMEM((WIN,), jnp.int32),
          pltpu.VMEM((WIN,), jnp.int32),
          pltpu.VMEM((WIN,), jnp.float32),
          pltpu.VMEM((WIN, H), jnp.float32),
          pltpu.SemaphoreType.DMA,
      ],
  )
  def msg_kernel(g_hbm, src_hbm, dst_hbm, ew_hbm, out_hbm,
                 acc, sidx, didx, ewv, rows, sem):
    cid = lax.axis_index("c")
    sid = lax.axis_index("s")
    wid = sid * NC + cid
    base = wid * epw
    r0 = sid * rpt
    pltpu.sync_copy(g_hbm.at[pl.ds(r0, rpt)], acc.at[pl.ds(r0, rpt)])
    if rem:
      @pl.when(sid == NS - 1)
      def _():
        pltpu.sync_copy(g_hbm.at[pl.ds(rpt * NS, rem)],
                        acc.at[pl.ds(rpt * NS, rem)])
    plsc.subcore_barrier()

    def win(w, carry):
      off = base + w * WIN
      pltpu.sync_copy(src_hbm.at[pl.ds(off, WIN)], sidx)
      pltpu.sync_copy(dst_hbm.at[pl.ds(off, WIN)], didx)
      pltpu.sync_copy(ew_hbm.at[pl.ds(off, WIN)], ewv)
      pltpu.async_copy(g_hbm.at[sidx], rows, sem).wait()

      def group(jg, c2):
        j0 = jg * LANES
        cvec = ewv[pl.ds(j0, LANES)]
        dn = lax.GatherDimensionNumbers(
            offset_dims=(), collapsed_slice_dims=(0,), start_index_map=(0,))
        for t in range(LANES):
          c = lax.gather(cvec, jnp.full((LANES, 1), t, jnp.int32), dn,
                         slice_sizes=(1,),
                         mode=lax.GatherScatterMode.PROMISE_IN_BOUNDS)
          for k in range(H // LANES):
            sl = pl.ds(k * LANES, LANES)
            rows[j0 + t, sl] = rows[j0 + t, sl] * c
        return c2

      lax.fori_loop(0, WIN // LANES, group, 0)
      pltpu.sync_copy(rows, acc.at[didx], add=True)
      return carry

    lax.fori_loop(0, nwin, win, 0)
    plsc.subcore_barrier()
    pltpu.sync_copy(acc.at[pl.ds(r0, rpt)], out_hbm.at[cid, pl.ds(r0, rpt)])
    if rem:
      @pl.when(sid == NS - 1)
      def _():
        pltpu.sync_copy(acc.at[pl.ds(rpt * NS, rem)],
                        out_hbm.at[cid, pl.ds(rpt * NS, rem)])

  return msg_kernel


def _tc_dinv_body(degp_ref, dinv_ref):
  deg = degp_ref[0, :] + degp_ref[1, :] + 1.0
  dinv_ref[...] = jnp.where(deg > 0, lax.rsqrt(deg), 0.0)


def _tc_mlp_body(N, x_ref, w1_ref, b1_ref, w2_ref, b2_ref, wc_ref, dinv_ref,
                 g_ref):
  h = jnp.maximum(jnp.dot(x_ref[...], w1_ref[...],
                          preferred_element_type=jnp.float32) + b1_ref[...],
                  0.0)
  h = jnp.maximum(jnp.dot(h, w2_ref[...],
                          preferred_element_type=jnp.float32) + b2_ref[...],
                  0.0)
  dinv = dinv_ref[pl.ds(0, N), :]
  g_ref[...] = jnp.dot(h, wc_ref[...],
                       preferred_element_type=jnp.float32) * dinv


def _tc_post_body(N, H, G, accp_ref, g_ref, dinv_ref, bc_ref, bat_ref,
                  wcn_ref, pool_ref, gnext_ref):
  dinv = dinv_ref[pl.ds(0, N), :]
  acc = accp_ref[0] + accp_ref[1] - g_ref[...]
  h = jnp.maximum(acc * dinv + bc_ref[...], 0.0)
  gnext_ref[...] = jnp.dot(h, wcn_ref[...],
                           preferred_element_type=jnp.float32) * dinv
  bat = bat_ref[...]
  onehot = (bat == lax.broadcasted_iota(jnp.int32, (1, G), 1)
            ).astype(jnp.float32)
  dnums = (((0,), (0,)), ((), ()))
  ssum = lax.dot_general(onehot, h, dnums,
                         preferred_element_type=jnp.float32)
  cnt = lax.dot_general(onehot, jnp.ones((N, 1), jnp.float32), dnums,
                        preferred_element_type=jnp.float32)
  pool_ref[:, pl.ds(H, H)] = ssum / jnp.maximum(cnt, 1.0)

  def body(g8, carry):
    rs = []
    for t in range(8):
      m = jnp.where(bat == g8 * 8 + t, h, -jnp.inf)
      rs.append(jnp.max(m, axis=0, keepdims=True))
    r = jnp.concatenate(rs, axis=0)
    r = jnp.where(jnp.abs(r) < jnp.inf, r, 0.0)
    pool_ref[pl.ds(pl.multiple_of(g8 * 8, 8), 8), pl.ds(0, H)] = r
    return carry

  lax.fori_loop(0, G // 8, body, 0)


def kernel(x, edge_index, edge_attr, batch, W1, b1, W2, b2,
           Wc0, bc0, Wc1, bc1, Wc2, bc2):
  N, DF = x.shape
  H = W1.shape[1]
  E = edge_index.shape[1]
  G = 64
  n_pad = ((N + NS * LANES - 1) // (NS * LANES)) * (NS * LANES)
  assert E % (NC * NS * WIN) == 0 and N % 8 == 0

  src = jnp.asarray(edge_index[0], jnp.int32)
  dst = jnp.asarray(edge_index[1], jnp.int32)
  ew = edge_attr.reshape(E)
  bat2 = batch.reshape(N, 1).astype(jnp.int32)

  degp = _sc_degree(E, n_pad)(dst, ew)

  dinv1 = pl.pallas_call(
      _tc_dinv_body,
      out_shape=jax.ShapeDtypeStruct((n_pad,), jnp.float32),
  )(degp)
  dinv2 = dinv1.reshape(n_pad, 1)

  g = pl.pallas_call(
      functools.partial(_tc_mlp_body, N),
      out_shape=jax.ShapeDtypeStruct((N, H), jnp.float32),
  )(x, W1, b1.reshape(1, H), W2, b2.reshape(1, H), Wc0, dinv2)

  msg = _sc_message(N, E, H)
  post = pl.pallas_call(
      functools.partial(_tc_post_body, N, H, G),
      out_shape=(jax.ShapeDtypeStruct((G, 2 * H), jnp.float32),
                 jax.ShapeDtypeStruct((N, H), jnp.float32)),
      compiler_params=pltpu.CompilerParams(
          vmem_limit_bytes=100 * 1024 * 1024),
  )

  pools = []
  for bc, wcn in ((bc0, Wc1), (bc1, Wc2), (bc2, Wc2)):
    accp = msg(g, src, dst, ew)
    pool, g = post(accp, g, dinv2, bc.reshape(1, H), bat2, wcn)
    pools.append(pool)

  return jnp.concatenate(pools, axis=1)


# speedup vs baseline: 14.8262x; 2.0227x over previous
import functools

import jax
import jax.numpy as jnp
from jax import lax
from jax.experimental import pallas as pl
from jax.experimental.pallas import tpu as pltpu
from jax.experimental.pallas import tpu_sc as plsc

NC = 2
NS = 16
LANES = 16
WIN = 80


def _mesh():
  return plsc.VectorSubcoreMesh(
      core_axis_name="c", subcore_axis_name="s", num_cores=NC,
      num_subcores=NS)


def _sc_degree(E, n_pad):
  epw = E // (NC * NS)
  nwin = epw // WIN
  rpt = n_pad // NS

  assert nwin % 2 == 1

  @functools.partial(
      pl.kernel,
      out_type=jax.ShapeDtypeStruct((NC, n_pad), jnp.float32),
      mesh=_mesh(),
      scratch_types=[
          pltpu.VMEM_SHARED((n_pad,), jnp.float32),
          pltpu.VMEM((rpt,), jnp.float32),
          pltpu.VMEM((epw,), jnp.float32),
          pltpu.VMEM((WIN,), jnp.int32),
          pltpu.VMEM((WIN,), jnp.int32),
          pltpu.SemaphoreType.DMA,
          pltpu.SemaphoreType.DMA,
      ],
  )
  def deg_kernel(dst_hbm, ew_hbm, out_hbm, acc, zbuf, ewv, dw0, dw1,
                 semd0, semd1):
    cid = lax.axis_index("c")
    sid = lax.axis_index("s")
    wid = sid * NC + cid
    base = wid * epw
    for i in range(rpt // LANES):
      zbuf[pl.ds(i * LANES, LANES)] = jnp.zeros((LANES,), jnp.float32)
    r0 = sid * rpt
    pltpu.sync_copy(zbuf, acc.at[pl.ds(r0, rpt)])
    pltpu.sync_copy(ew_hbm.at[pl.ds(base, epw)], ewv)
    pltpu.async_copy(dst_hbm.at[pl.ds(base, WIN)], dw0, semd0)
    plsc.subcore_barrier()

    def pair(wb, carry):
      w0 = wb * 2
      o0 = base + w0 * WIN
      pltpu.async_copy(dst_hbm.at[pl.ds(o0 + WIN, WIN)], dw1, semd1)
      pltpu.make_async_copy(dst_hbm.at[pl.ds(o0, WIN)], dw0, semd0).wait()
      pltpu.sync_copy(ewv.at[pl.ds(w0 * WIN, WIN)], acc.at[dw0], add=True)
      pltpu.async_copy(dst_hbm.at[pl.ds(o0 + 2 * WIN, WIN)], dw0, semd0)
      pltpu.make_async_copy(dst_hbm.at[pl.ds(o0 + WIN, WIN)], dw1,
                            semd1).wait()
      pltpu.sync_copy(ewv.at[pl.ds((w0 + 1) * WIN, WIN)], acc.at[dw1],
                      add=True)
      return carry

    lax.fori_loop(0, (nwin - 1) // 2, pair, 0)
    pltpu.make_async_copy(dst_hbm.at[pl.ds(base + (nwin - 1) * WIN, WIN)],
                          dw0, semd0).wait()
    pltpu.sync_copy(ewv.at[pl.ds((nwin - 1) * WIN, WIN)], acc.at[dw0],
                    add=True)
    plsc.subcore_barrier()
    pltpu.sync_copy(acc.at[pl.ds(r0, rpt)], out_hbm.at[cid, pl.ds(r0, rpt)])

  return deg_kernel


def _sc_message(N, E, H):
  epw = E // (NC * NS)
  nwin = epw // WIN
  assert nwin % 2 == 1
  rpt = (N // (8 * NS)) * 8
  rem = N - rpt * NS

  @functools.partial(
      pl.kernel,
      out_type=jax.ShapeDtypeStruct((NC, N, H), jnp.float32),
      mesh=_mesh(),
      scratch_types=[
          pltpu.VMEM_SHARED((N, H), jnp.float32),
          pltpu.VMEM((epw,), jnp.int32),
          pltpu.VMEM((WIN,), jnp.int32),
          pltpu.VMEM((WIN,), jnp.int32),
          pltpu.VMEM((epw,), jnp.float32),
          pltpu.VMEM((WIN, H), jnp.float32),
          pltpu.VMEM((WIN, H), jnp.float32),
          pltpu.SemaphoreType.DMA,
          pltpu.SemaphoreType.DMA,
          pltpu.SemaphoreType.DMA,
          pltpu.SemaphoreType.DMA,
      ],
  )
  def msg_kernel(g_hbm, src_hbm, dst_hbm, ew_hbm, out_hbm,
                 acc, sidx, dw0, dw1, ewv, rows0, rows1,
                 sem0, sem1, semd0, semd1):
    cid = lax.axis_index("c")
    sid = lax.axis_index("s")
    wid = sid * NC + cid
    base = wid * epw
    r0 = sid * rpt
    pltpu.sync_copy(src_hbm.at[pl.ds(base, epw)], sidx)
    pltpu.sync_copy(ew_hbm.at[pl.ds(base, epw)], ewv)
    pltpu.async_copy(dst_hbm.at[pl.ds(base, WIN)], dw0, semd0)
    pltpu.async_copy(g_hbm.at[sidx.at[pl.ds(0, WIN)]], rows0, sem0)
    pltpu.sync_copy(g_hbm.at[pl.ds(r0, rpt)], acc.at[pl.ds(r0, rpt)])
    if rem:
      @pl.when(sid == NS - 1)
      def _():
        pltpu.sync_copy(g_hbm.at[pl.ds(rpt * NS, rem)],
                        acc.at[pl.ds(rpt * NS, rem)])
    plsc.subcore_barrier()

    dn = lax.GatherDimensionNumbers(
        offset_dims=(), collapsed_slice_dims=(0,), start_index_map=(0,))

    def scale(rows, w):
      for jg in range(WIN // LANES):
        j0 = jg * LANES
        cvec = ewv[pl.ds(w * WIN + j0, LANES)]
        for t in range(LANES):
          c = lax.gather(cvec, jnp.full((LANES, 1), t, jnp.int32), dn,
                         slice_sizes=(1,),
                         mode=lax.GatherScatterMode.PROMISE_IN_BOUNDS)
          for k in range(H // LANES):
            sl = pl.ds(k * LANES, LANES)
            rows[j0 + t, sl] = rows[j0 + t, sl] * c

    def swin(w):
      return sidx.at[pl.ds(w * WIN, WIN)]

    def dwin(w):
      return dst_hbm.at[pl.ds(base + w * WIN, WIN)]

    def pair(wb, carry):
      w0 = wb * 2
      pltpu.async_copy(dwin(w0 + 1), dw1, semd1)
      pltpu.async_copy(g_hbm.at[swin(w0 + 1)], rows1, sem1)
      pltpu.make_async_copy(g_hbm.at[swin(w0)], rows0, sem0).wait()
      scale(rows0, w0)
      pltpu.make_async_copy(dwin(w0), dw0, semd0).wait()
      pltpu.sync_copy(rows0, acc.at[dw0], add=True)
      pltpu.async_copy(dwin(w0 + 2), dw0, semd0)
      pltpu.async_copy(g_hbm.at[swin(w0 + 2)], rows0, sem0)
      pltpu.make_async_copy(g_hbm.at[swin(w0 + 1)], rows1, sem1).wait()
      scale(rows1, w0 + 1)
      pltpu.make_async_copy(dwin(w0 + 1), dw1, semd1).wait()
      pltpu.sync_copy(rows1, acc.at[dw1], add=True)
      return carry

    lax.fori_loop(0, (nwin - 1) // 2, pair, 0)
    pltpu.make_async_copy(g_hbm.at[swin(nwin - 1)], rows0, sem0).wait()
    scale(rows0, nwin - 1)
    pltpu.make_async_copy(dwin(nwin - 1), dw0, semd0).wait()
    pltpu.sync_copy(rows0, acc.at[dw0], add=True)

    plsc.subcore_barrier()
    pltpu.sync_copy(acc.at[pl.ds(r0, rpt)], out_hbm.at[cid, pl.ds(r0, rpt)])
    if rem:
      @pl.when(sid == NS - 1)
      def _():
        pltpu.sync_copy(acc.at[pl.ds(rpt * NS, rem)],
                        out_hbm.at[cid, pl.ds(rpt * NS, rem)])

  return msg_kernel


def _tc_dinv_body(degp_ref, dinv_ref):
  deg = degp_ref[0, :] + degp_ref[1, :] + 1.0
  dinv_ref[...] = jnp.where(deg > 0, lax.rsqrt(deg), 0.0)


def _tc_mlp_body(N, x_ref, w1_ref, b1_ref, w2_ref, b2_ref, wc_ref, dinv_ref,
                 g_ref):
  h = jnp.maximum(jnp.dot(x_ref[...], w1_ref[...],
                          preferred_element_type=jnp.float32) + b1_ref[...],
                  0.0)
  h = jnp.maximum(jnp.dot(h, w2_ref[...],
                          preferred_element_type=jnp.float32) + b2_ref[...],
                  0.0)
  dinv = dinv_ref[pl.ds(0, N), :]
  g_ref[...] = jnp.dot(h, wc_ref[...],
                       preferred_element_type=jnp.float32) * dinv


def _tc_post_body(N, H, G, accp_ref, g_ref, dinv_ref, bc_ref, bat_ref,
                  wcn_ref, pool_ref, gnext_ref):
  dinv = dinv_ref[pl.ds(0, N), :]
  acc = accp_ref[0] + accp_ref[1] - g_ref[...]
  h = jnp.maximum(acc * dinv + bc_ref[...], 0.0)
  gnext_ref[...] = jnp.dot(h, wcn_ref[...],
                           preferred_element_type=jnp.float32) * dinv
  bat = bat_ref[...]
  onehot = (bat == lax.broadcasted_iota(jnp.int32, (1, G), 1)
            ).astype(jnp.float32)
  dnums = (((0,), (0,)), ((), ()))
  ssum = lax.dot_general(onehot, h, dnums,
                         preferred_element_type=jnp.float32)
  cnt = lax.dot_general(onehot, jnp.ones((N, 1), jnp.float32), dnums,
                        preferred_element_type=jnp.float32)
  pool_ref[:, pl.ds(H, H)] = ssum / jnp.maximum(cnt, 1.0)

  def body(g8, carry):
    rs = []
    for t in range(8):
      m = jnp.where(bat == g8 * 8 + t, h, -jnp.inf)
      rs.append(jnp.max(m, axis=0, keepdims=True))
    r = jnp.concatenate(rs, axis=0)
    r = jnp.where(jnp.abs(r) < jnp.inf, r, 0.0)
    pool_ref[pl.ds(pl.multiple_of(g8 * 8, 8), 8), pl.ds(0, H)] = r
    return carry

  lax.fori_loop(0, G // 8, body, 0)


def kernel(x, edge_index, edge_attr, batch, W1, b1, W2, b2,
           Wc0, bc0, Wc1, bc1, Wc2, bc2):
  N, DF = x.shape
  H = W1.shape[1]
  E = edge_index.shape[1]
  G = 64
  n_pad = ((N + NS * LANES - 1) // (NS * LANES)) * (NS * LANES)
  assert E % (NC * NS * WIN) == 0 and N % 8 == 0

  src = jnp.asarray(edge_index[0], jnp.int32)
  dst = jnp.asarray(edge_index[1], jnp.int32)
  ew = edge_attr.reshape(E)
  bat2 = batch.reshape(N, 1).astype(jnp.int32)

  degp = _sc_degree(E, n_pad)(dst, ew)

  dinv1 = pl.pallas_call(
      _tc_dinv_body,
      out_shape=jax.ShapeDtypeStruct((n_pad,), jnp.float32),
  )(degp)
  dinv2 = dinv1.reshape(n_pad, 1)

  g = pl.pallas_call(
      functools.partial(_tc_mlp_body, N),
      out_shape=jax.ShapeDtypeStruct((N, H), jnp.float32),
  )(x, W1, b1.reshape(1, H), W2, b2.reshape(1, H), Wc0, dinv2)

  msg = _sc_message(N, E, H)
  post = pl.pallas_call(
      functools.partial(_tc_post_body, N, H, G),
      out_shape=(jax.ShapeDtypeStruct((G, 2 * H), jnp.float32),
                 jax.ShapeDtypeStruct((N, H), jnp.float32)),
      compiler_params=pltpu.CompilerParams(
          vmem_limit_bytes=100 * 1024 * 1024),
  )

  pools = []
  for bc, wcn in ((bc0, Wc1), (bc1, Wc2), (bc2, Wc2)):
    accp = msg(g, src, dst, ew)
    pool, g = post(accp, g, dinv2, bc.reshape(1, H), bat2, wcn)
    pools.append(pool)

  return jnp.concatenate(pools, axis=1)


# speedup vs baseline: 17.5148x; 1.1813x over previous
import functools

import jax
import jax.numpy as jnp
from jax import lax
from jax.experimental import pallas as pl
from jax.experimental.pallas import tpu as pltpu
from jax.experimental.pallas import tpu_sc as plsc

NC = 2
NS = 16
LANES = 16
WIN = 80


def _mesh():
  return plsc.VectorSubcoreMesh(
      core_axis_name="c", subcore_axis_name="s", num_cores=NC,
      num_subcores=NS)


def _sc_degree(E, n_pad):
  epw = E // (NC * NS)
  nwin = epw // WIN
  rpt = n_pad // NS

  assert nwin % 2 == 1

  @functools.partial(
      pl.kernel,
      out_type=jax.ShapeDtypeStruct((NC, n_pad), jnp.float32),
      mesh=_mesh(),
      scratch_types=[
          pltpu.VMEM_SHARED((n_pad,), jnp.float32),
          pltpu.VMEM((rpt,), jnp.float32),
          pltpu.VMEM((epw,), jnp.float32),
          pltpu.VMEM((WIN,), jnp.int32),
          pltpu.VMEM((WIN,), jnp.int32),
          pltpu.SemaphoreType.DMA,
          pltpu.SemaphoreType.DMA,
      ],
  )
  def deg_kernel(dst_hbm, ew_hbm, out_hbm, acc, zbuf, ewv, dw0, dw1,
                 semd0, semd1):
    cid = lax.axis_index("c")
    sid = lax.axis_index("s")
    wid = sid * NC + cid
    base = wid * epw
    for i in range(rpt // LANES):
      zbuf[pl.ds(i * LANES, LANES)] = jnp.zeros((LANES,), jnp.float32)
    r0 = sid * rpt
    pltpu.sync_copy(zbuf, acc.at[pl.ds(r0, rpt)])
    pltpu.sync_copy(ew_hbm.at[pl.ds(base, epw)], ewv)
    pltpu.async_copy(dst_hbm.at[pl.ds(base, WIN)], dw0, semd0)
    plsc.subcore_barrier()

    def pair(wb, carry):
      w0 = wb * 2
      o0 = base + w0 * WIN
      pltpu.async_copy(dst_hbm.at[pl.ds(o0 + WIN, WIN)], dw1, semd1)
      pltpu.make_async_copy(dst_hbm.at[pl.ds(o0, WIN)], dw0, semd0).wait()
      pltpu.sync_copy(ewv.at[pl.ds(w0 * WIN, WIN)], acc.at[dw0], add=True)
      pltpu.async_copy(dst_hbm.at[pl.ds(o0 + 2 * WIN, WIN)], dw0, semd0)
      pltpu.make_async_copy(dst_hbm.at[pl.ds(o0 + WIN, WIN)], dw1,
                            semd1).wait()
      pltpu.sync_copy(ewv.at[pl.ds((w0 + 1) * WIN, WIN)], acc.at[dw1],
                      add=True)
      return carry

    lax.fori_loop(0, (nwin - 1) // 2, pair, 0)
    pltpu.make_async_copy(dst_hbm.at[pl.ds(base + (nwin - 1) * WIN, WIN)],
                          dw0, semd0).wait()
    pltpu.sync_copy(ewv.at[pl.ds((nwin - 1) * WIN, WIN)], acc.at[dw0],
                    add=True)
    plsc.subcore_barrier()
    pltpu.sync_copy(acc.at[pl.ds(r0, rpt)], out_hbm.at[cid, pl.ds(r0, rpt)])

  return deg_kernel


def _sc_message(N, E, H):
  epw = E // (NC * NS)
  nwin = epw // WIN
  assert nwin % 2 == 1
  rpt = (N // (8 * NS)) * 8
  rem = N - rpt * NS

  @functools.partial(
      pl.kernel,
      out_type=jax.ShapeDtypeStruct((NC, N, H), jnp.float32),
      mesh=_mesh(),
      scratch_types=[
          pltpu.VMEM_SHARED((N, H), jnp.float32),
          pltpu.VMEM((epw,), jnp.int32),
          pltpu.VMEM((WIN,), jnp.int32),
          pltpu.VMEM((WIN,), jnp.int32),
          pltpu.VMEM((epw,), jnp.float32),
          pltpu.VMEM((WIN, H), jnp.float32),
          pltpu.VMEM((WIN, H), jnp.float32),
          pltpu.SemaphoreType.DMA,
          pltpu.SemaphoreType.DMA,
          pltpu.SemaphoreType.DMA,
          pltpu.SemaphoreType.DMA,
      ],
  )
  def msg_kernel(g_hbm, src_hbm, dst_hbm, ew_hbm, out_hbm,
                 acc, sidx, dw0, dw1, ewv, rows0, rows1,
                 sem0, sem1, semd0, semd1):
    cid = lax.axis_index("c")
    sid = lax.axis_index("s")
    wid = sid * NC + cid
    base = wid * epw
    r0 = sid * rpt
    pltpu.sync_copy(src_hbm.at[pl.ds(base, epw)], sidx)
    pltpu.sync_copy(ew_hbm.at[pl.ds(base, epw)], ewv)
    pltpu.async_copy(dst_hbm.at[pl.ds(base, WIN)], dw0, semd0)
    pltpu.async_copy(g_hbm.at[sidx.at[pl.ds(0, WIN)]], rows0, sem0)
    pltpu.sync_copy(g_hbm.at[pl.ds(r0, rpt)], acc.at[pl.ds(r0, rpt)])
    if rem:
      @pl.when(sid == NS - 1)
      def _():
        pltpu.sync_copy(g_hbm.at[pl.ds(rpt * NS, rem)],
                        acc.at[pl.ds(rpt * NS, rem)])
    plsc.subcore_barrier()

    dn = lax.GatherDimensionNumbers(
        offset_dims=(), collapsed_slice_dims=(0,), start_index_map=(0,))

    def scale(rows, w):
      for jg in range(WIN // LANES):
        j0 = jg * LANES
        cvec = ewv[pl.ds(w * WIN + j0, LANES)]
        for t in range(LANES):
          c = lax.gather(cvec, jnp.full((LANES, 1), t, jnp.int32), dn,
                         slice_sizes=(1,),
                         mode=lax.GatherScatterMode.PROMISE_IN_BOUNDS)
          for k in range(H // LANES):
            sl = pl.ds(k * LANES, LANES)
            rows[j0 + t, sl] = rows[j0 + t, sl] * c

    def swin(w):
      return sidx.at[pl.ds(w * WIN, WIN)]

    def dwin(w):
      return dst_hbm.at[pl.ds(base + w * WIN, WIN)]

    def pair(wb, carry):
      w0 = wb * 2
      pltpu.async_copy(dwin(w0 + 1), dw1, semd1)
      pltpu.async_copy(g_hbm.at[swin(w0 + 1)], rows1, sem1)
      pltpu.make_async_copy(g_hbm.at[swin(w0)], rows0, sem0).wait()
      scale(rows0, w0)
      pltpu.make_async_copy(dwin(w0), dw0, semd0).wait()
      pltpu.sync_copy(rows0, acc.at[dw0], add=True)
      pltpu.async_copy(dwin(w0 + 2), dw0, semd0)
      pltpu.async_copy(g_hbm.at[swin(w0 + 2)], rows0, sem0)
      pltpu.make_async_copy(g_hbm.at[swin(w0 + 1)], rows1, sem1).wait()
      scale(rows1, w0 + 1)
      pltpu.make_async_copy(dwin(w0 + 1), dw1, semd1).wait()
      pltpu.sync_copy(rows1, acc.at[dw1], add=True)
      return carry

    lax.fori_loop(0, (nwin - 1) // 2, pair, 0)
    pltpu.make_async_copy(g_hbm.at[swin(nwin - 1)], rows0, sem0).wait()
    scale(rows0, nwin - 1)
    pltpu.make_async_copy(dwin(nwin - 1), dw0, semd0).wait()
    pltpu.sync_copy(rows0, acc.at[dw0], add=True)

    plsc.subcore_barrier()
    pltpu.sync_copy(acc.at[pl.ds(r0, rpt)], out_hbm.at[cid, pl.ds(r0, rpt)])
    if rem:
      @pl.when(sid == NS - 1)
      def _():
        pltpu.sync_copy(acc.at[pl.ds(rpt * NS, rem)],
                        out_hbm.at[cid, pl.ds(rpt * NS, rem)])

  return msg_kernel


def _tc_dinv_body(degp_ref, dinv_ref):
  deg = degp_ref[0, :] + degp_ref[1, :] + 1.0
  dinv_ref[...] = jnp.where(deg > 0, lax.rsqrt(deg), 0.0)


def _tc_mlp_body(x_ref, w1_ref, b1_ref, w2_ref, b2_ref, h_ref):
  h = jnp.maximum(jnp.dot(x_ref[...], w1_ref[...],
                          preferred_element_type=jnp.float32) + b1_ref[...],
                  0.0)
  h_ref[...] = jnp.maximum(
      jnp.dot(h, w2_ref[...], preferred_element_type=jnp.float32)
      + b2_ref[...], 0.0)


def _tc_scale_body(N, h_ref, wc_ref, dinv_ref, g_ref):
  dinv = dinv_ref[pl.ds(0, N), :]
  g_ref[...] = jnp.dot(h_ref[...], wc_ref[...],
                       preferred_element_type=jnp.float32) * dinv


def _tc_combine_body(N, accp_ref, g_ref, dinv_ref, bc_ref, wcn_ref,
                     h_ref, gnext_ref):
  dinv = dinv_ref[pl.ds(0, N), :]
  acc = accp_ref[0] + accp_ref[1] - g_ref[...]
  h = jnp.maximum(acc * dinv + bc_ref[...], 0.0)
  h_ref[...] = h
  gnext_ref[...] = jnp.dot(h, wcn_ref[...],
                           preferred_element_type=jnp.float32) * dinv


def _tc_combine_last_body(N, accp_ref, g_ref, dinv_ref, bc_ref, h_ref):
  dinv = dinv_ref[pl.ds(0, N), :]
  acc = accp_ref[0] + accp_ref[1] - g_ref[...]
  h_ref[...] = jnp.maximum(acc * dinv + bc_ref[...], 0.0)


def _tc_pool_body(N, H, G, h_ref, bat_ref, pool_ref):
  h = h_ref[...]
  bat = bat_ref[...]
  onehot = (bat == lax.broadcasted_iota(jnp.int32, (1, G), 1)
            ).astype(jnp.float32)
  dnums = (((0,), (0,)), ((), ()))
  ssum = lax.dot_general(onehot, h, dnums,
                         preferred_element_type=jnp.float32)
  cnt = lax.dot_general(onehot, jnp.ones((N, 1), jnp.float32), dnums,
                        preferred_element_type=jnp.float32)
  pool_ref[:, pl.ds(H, H)] = ssum / jnp.maximum(cnt, 1.0)

  def body(g8, carry):
    rs = []
    for t in range(8):
      m = jnp.where(bat == g8 * 8 + t, h, -jnp.inf)
      rs.append(jnp.max(m, axis=0, keepdims=True))
    r = jnp.concatenate(rs, axis=0)
    r = jnp.where(jnp.abs(r) < jnp.inf, r, 0.0)
    pool_ref[pl.ds(pl.multiple_of(g8 * 8, 8), 8), pl.ds(0, H)] = r
    return carry

  lax.fori_loop(0, G // 8, body, 0)


def kernel(x, edge_index, edge_attr, batch, W1, b1, W2, b2,
           Wc0, bc0, Wc1, bc1, Wc2, bc2):
  N, DF = x.shape
  H = W1.shape[1]
  E = edge_index.shape[1]
  G = 64
  n_pad = ((N + NS * LANES - 1) // (NS * LANES)) * (NS * LANES)
  assert E % (NC * NS * WIN) == 0 and N % 8 == 0

  src = jnp.asarray(edge_index[0], jnp.int32)
  dst = jnp.asarray(edge_index[1], jnp.int32)
  ew = edge_attr.reshape(E)
  bat2 = batch.reshape(N, 1).astype(jnp.int32)

  vmem100 = pltpu.CompilerParams(vmem_limit_bytes=100 * 1024 * 1024)
  nh = jax.ShapeDtypeStruct((N, H), jnp.float32)

  degp = _sc_degree(E, n_pad)(dst, ew)

  h2 = pl.pallas_call(
      _tc_mlp_body, out_shape=nh, compiler_params=vmem100,
  )(x, W1, b1.reshape(1, H), W2, b2.reshape(1, H))

  dinv1 = pl.pallas_call(
      _tc_dinv_body,
      out_shape=jax.ShapeDtypeStruct((n_pad,), jnp.float32),
  )(degp)
  dinv2 = dinv1.reshape(n_pad, 1)

  g = pl.pallas_call(
      functools.partial(_tc_scale_body, N),
      out_shape=nh, compiler_params=vmem100,
  )(h2, Wc0, dinv2)

  msg = _sc_message(N, E, H)
  combine = pl.pallas_call(
      functools.partial(_tc_combine_body, N),
      out_shape=(nh, nh), compiler_params=vmem100,
  )
  combine_last = pl.pallas_call(
      functools.partial(_tc_combine_last_body, N),
      out_shape=nh, compiler_params=vmem100,
  )
  pool_call = pl.pallas_call(
      functools.partial(_tc_pool_body, N, H, G),
      out_shape=jax.ShapeDtypeStruct((G, 2 * H), jnp.float32),
      compiler_params=vmem100,
  )

  pools = []
  for li, (bc, wcn) in enumerate(((bc0, Wc1), (bc1, Wc2), (bc2, None))):
    accp = msg(g, src, dst, ew)
    if wcn is None:
      h = combine_last(accp, g, dinv2, bc.reshape(1, H))
    else:
      h, g = combine(accp, g, dinv2, bc.reshape(1, H), wcn)
    pools.append(pool_call(h, bat2))

  return jnp.concatenate(pools, axis=1)


# speedup vs baseline: 17.5481x; 1.0019x over previous
import functools

import jax
import jax.numpy as jnp
from jax import lax
from jax.experimental import pallas as pl
from jax.experimental.pallas import tpu as pltpu
from jax.experimental.pallas import tpu_sc as plsc

NC = 2
NS = 16
LANES = 16
WIN = 80


def _mesh():
  return plsc.VectorSubcoreMesh(
      core_axis_name="c", subcore_axis_name="s", num_cores=NC,
      num_subcores=NS)


def _sc_degree(E, n_pad):
  epw = E // (NC * NS)
  nwin = epw // WIN
  rpt = n_pad // NS

  assert nwin % 2 == 1

  @functools.partial(
      pl.kernel,
      out_type=jax.ShapeDtypeStruct((NC, n_pad), jnp.float32),
      mesh=_mesh(),
      scratch_types=[
          pltpu.VMEM_SHARED((n_pad,), jnp.float32),
          pltpu.VMEM((rpt,), jnp.float32),
          pltpu.VMEM((epw,), jnp.float32),
          pltpu.VMEM((WIN,), jnp.int32),
          pltpu.VMEM((WIN,), jnp.int32),
          pltpu.SemaphoreType.DMA,
          pltpu.SemaphoreType.DMA,
      ],
  )
  def deg_kernel(dst_hbm, ew_hbm, out_hbm, acc, zbuf, ewv, dw0, dw1,
                 semd0, semd1):
    cid = lax.axis_index("c")
    sid = lax.axis_index("s")
    wid = sid * NC + cid
    base = wid * epw
    for i in range(rpt // LANES):
      zbuf[pl.ds(i * LANES, LANES)] = jnp.zeros((LANES,), jnp.float32)
    r0 = sid * rpt
    pltpu.sync_copy(zbuf, acc.at[pl.ds(r0, rpt)])
    pltpu.sync_copy(ew_hbm.at[pl.ds(base, epw)], ewv)
    pltpu.async_copy(dst_hbm.at[pl.ds(base, WIN)], dw0, semd0)
    plsc.subcore_barrier()

    def pair(wb, carry):
      w0 = wb * 2
      o0 = base + w0 * WIN
      pltpu.async_copy(dst_hbm.at[pl.ds(o0 + WIN, WIN)], dw1, semd1)
      pltpu.make_async_copy(dst_hbm.at[pl.ds(o0, WIN)], dw0, semd0).wait()
      pltpu.sync_copy(ewv.at[pl.ds(w0 * WIN, WIN)], acc.at[dw0], add=True)
      pltpu.async_copy(dst_hbm.at[pl.ds(o0 + 2 * WIN, WIN)], dw0, semd0)
      pltpu.make_async_copy(dst_hbm.at[pl.ds(o0 + WIN, WIN)], dw1,
                            semd1).wait()
      pltpu.sync_copy(ewv.at[pl.ds((w0 + 1) * WIN, WIN)], acc.at[dw1],
                      add=True)
      return carry

    lax.fori_loop(0, (nwin - 1) // 2, pair, 0)
    pltpu.make_async_copy(dst_hbm.at[pl.ds(base + (nwin - 1) * WIN, WIN)],
                          dw0, semd0).wait()
    pltpu.sync_copy(ewv.at[pl.ds((nwin - 1) * WIN, WIN)], acc.at[dw0],
                    add=True)
    plsc.subcore_barrier()
    pltpu.sync_copy(acc.at[pl.ds(r0, rpt)], out_hbm.at[cid, pl.ds(r0, rpt)])

  return deg_kernel


def _sc_message(N, E, H):
  epw = E // (NC * NS)
  nwin = epw // WIN
  assert nwin % 2 == 1
  rpt = (N // (8 * NS)) * 8
  rem = N - rpt * NS

  @functools.partial(
      pl.kernel,
      out_type=jax.ShapeDtypeStruct((NC, N, H), jnp.float32),
      mesh=_mesh(),
      scratch_types=[
          pltpu.VMEM_SHARED((N, H), jnp.float32),
          pltpu.VMEM((epw,), jnp.int32),
          pltpu.VMEM((WIN,), jnp.int32),
          pltpu.VMEM((WIN,), jnp.int32),
          pltpu.VMEM((epw,), jnp.float32),
          pltpu.VMEM((WIN, H), jnp.float32),
          pltpu.VMEM((WIN, H), jnp.float32),
          pltpu.SemaphoreType.DMA,
          pltpu.SemaphoreType.DMA,
          pltpu.SemaphoreType.DMA,
          pltpu.SemaphoreType.DMA,
      ],
  )
  def msg_kernel(g_hbm, src_hbm, dst_hbm, ew_hbm, out_hbm,
                 acc, sidx, dw0, dw1, ewv, rows0, rows1,
                 sem0, sem1, semd0, semd1):
    cid = lax.axis_index("c")
    sid = lax.axis_index("s")
    wid = sid * NC + cid
    base = wid * epw
    r0 = sid * rpt
    pltpu.sync_copy(src_hbm.at[pl.ds(base, epw)], sidx)
    pltpu.sync_copy(ew_hbm.at[pl.ds(base, epw)], ewv)
    pltpu.async_copy(dst_hbm.at[pl.ds(base, WIN)], dw0, semd0)
    pltpu.async_copy(g_hbm.at[sidx.at[pl.ds(0, WIN)]], rows0, sem0)
    pltpu.sync_copy(g_hbm.at[pl.ds(r0, rpt)], acc.at[pl.ds(r0, rpt)])
    if rem:
      @pl.when(sid == NS - 1)
      def _():
        pltpu.sync_copy(g_hbm.at[pl.ds(rpt * NS, rem)],
                        acc.at[pl.ds(rpt * NS, rem)])
    plsc.subcore_barrier()

    dn = lax.GatherDimensionNumbers(
        offset_dims=(), collapsed_slice_dims=(0,), start_index_map=(0,))

    def scale(rows, w):
      for jg in range(WIN // LANES):
        j0 = jg * LANES
        cvec = ewv[pl.ds(w * WIN + j0, LANES)]
        for t in range(LANES):
          c = lax.gather(cvec, jnp.full((LANES, 1), t, jnp.int32), dn,
                         slice_sizes=(1,),
                         mode=lax.GatherScatterMode.PROMISE_IN_BOUNDS)
          for k in range(H // LANES):
            sl = pl.ds(k * LANES, LANES)
            rows[j0 + t, sl] = rows[j0 + t, sl] * c

    def swin(w):
      return sidx.at[pl.ds(w * WIN, WIN)]

    def dwin(w):
      return dst_hbm.at[pl.ds(base + w * WIN, WIN)]

    def pair(wb, carry):
      w0 = wb * 2
      pltpu.async_copy(dwin(w0 + 1), dw1, semd1)
      pltpu.async_copy(g_hbm.at[swin(w0 + 1)], rows1, sem1)
      pltpu.make_async_copy(g_hbm.at[swin(w0)], rows0, sem0).wait()
      scale(rows0, w0)
      pltpu.make_async_copy(dwin(w0), dw0, semd0).wait()
      pltpu.sync_copy(rows0, acc.at[dw0], add=True)
      pltpu.async_copy(dwin(w0 + 2), dw0, semd0)
      pltpu.async_copy(g_hbm.at[swin(w0 + 2)], rows0, sem0)
      pltpu.make_async_copy(g_hbm.at[swin(w0 + 1)], rows1, sem1).wait()
      scale(rows1, w0 + 1)
      pltpu.make_async_copy(dwin(w0 + 1), dw1, semd1).wait()
      pltpu.sync_copy(rows1, acc.at[dw1], add=True)
      return carry

    lax.fori_loop(0, (nwin - 1) // 2, pair, 0)
    pltpu.make_async_copy(g_hbm.at[swin(nwin - 1)], rows0, sem0).wait()
    scale(rows0, nwin - 1)
    pltpu.make_async_copy(dwin(nwin - 1), dw0, semd0).wait()
    pltpu.sync_copy(rows0, acc.at[dw0], add=True)

    plsc.subcore_barrier()
    pltpu.sync_copy(acc.at[pl.ds(r0, rpt)], out_hbm.at[cid, pl.ds(r0, rpt)])
    if rem:
      @pl.when(sid == NS - 1)
      def _():
        pltpu.sync_copy(acc.at[pl.ds(rpt * NS, rem)],
                        out_hbm.at[cid, pl.ds(rpt * NS, rem)])

  return msg_kernel


def _tc_dinv_body(degp_ref, dinv_ref):
  deg = degp_ref[0, :] + degp_ref[1, :] + 1.0
  dinv_ref[...] = jnp.where(deg > 0, lax.rsqrt(deg), 0.0)


def _tc_mlp_body(x_ref, w1_ref, b1_ref, w2_ref, b2_ref, h_ref):
  h = jnp.maximum(jnp.dot(x_ref[...], w1_ref[...],
                          preferred_element_type=jnp.float32) + b1_ref[...],
                  0.0)
  h_ref[...] = jnp.maximum(
      jnp.dot(h, w2_ref[...], preferred_element_type=jnp.float32)
      + b2_ref[...], 0.0)


def _tc_scale_body(N, h_ref, wc_ref, dinv_ref, g_ref):
  dinv = dinv_ref[pl.ds(0, N), :]
  g_ref[...] = jnp.dot(h_ref[...], wc_ref[...],
                       preferred_element_type=jnp.float32) * dinv


def _tc_combine_body(N, accp_ref, g_ref, dinv_ref, bc_ref, wcn_ref,
                     h_ref, gnext_ref):
  dinv = dinv_ref[pl.ds(0, N), :]
  acc = accp_ref[0] + accp_ref[1] - g_ref[...]
  h = jnp.maximum(acc * dinv + bc_ref[...], 0.0)
  h_ref[...] = h
  gnext_ref[...] = jnp.dot(h, wcn_ref[...],
                           preferred_element_type=jnp.float32) * dinv


def _tc_combine_last_body(N, accp_ref, g_ref, dinv_ref, bc_ref, h_ref):
  dinv = dinv_ref[pl.ds(0, N), :]
  acc = accp_ref[0] + accp_ref[1] - g_ref[...]
  h_ref[...] = jnp.maximum(acc * dinv + bc_ref[...], 0.0)


def _tc_pool_body(N, H, G, h_ref, bat_ref, pool_ref):
  h = h_ref[...]
  bat = bat_ref[...]
  onehot = (bat == lax.broadcasted_iota(jnp.int32, (1, G), 1)
            ).astype(jnp.float32)
  dnums = (((0,), (0,)), ((), ()))
  ssum = lax.dot_general(onehot, h, dnums,
                         preferred_element_type=jnp.float32)
  cnt = lax.dot_general(onehot, jnp.ones((N, 1), jnp.float32), dnums,
                        preferred_element_type=jnp.float32)
  pool_ref[:, pl.ds(H, H)] = ssum / jnp.maximum(cnt, 1.0)

  def body(g8, carry):
    rs = []
    for t in range(8):
      m = jnp.where(bat == g8 * 8 + t, h, -jnp.inf)
      rs.append(jnp.max(m, axis=0, keepdims=True))
    r = jnp.concatenate(rs, axis=0)
    r = jnp.where(jnp.abs(r) < jnp.inf, r, 0.0)
    pool_ref[pl.ds(pl.multiple_of(g8 * 8, 8), 8), pl.ds(0, H)] = r
    return carry

  lax.fori_loop(0, G // 8, body, 0)


def kernel(x, edge_index, edge_attr, batch, W1, b1, W2, b2,
           Wc0, bc0, Wc1, bc1, Wc2, bc2):
  N, DF = x.shape
  H = W1.shape[1]
  E = edge_index.shape[1]
  G = 64
  n_pad = ((N + NS * LANES - 1) // (NS * LANES)) * (NS * LANES)
  assert E % (NC * NS * WIN) == 0 and N % 8 == 0

  src = jnp.asarray(edge_index[0], jnp.int32)
  dst = jnp.asarray(edge_index[1], jnp.int32)
  ew = edge_attr.reshape(E)
  bat2 = batch.reshape(N, 1).astype(jnp.int32)

  vmem100 = pltpu.CompilerParams(vmem_limit_bytes=100 * 1024 * 1024)
  nh = jax.ShapeDtypeStruct((N, H), jnp.float32)

  degp = _sc_degree(E, n_pad)(dst, ew)

  h2 = pl.pallas_call(
      _tc_mlp_body, out_shape=nh, compiler_params=vmem100,
  )(x, W1, b1.reshape(1, H), W2, b2.reshape(1, H))

  dinv1 = pl.pallas_call(
      _tc_dinv_body,
      out_shape=jax.ShapeDtypeStruct((n_pad,), jnp.float32),
  )(degp)
  dinv2 = dinv1.reshape(n_pad, 1)

  g = pl.pallas_call(
      functools.partial(_tc_scale_body, N),
      out_shape=nh, compiler_params=vmem100,
  )(h2, Wc0, dinv2)

  msg = _sc_message(N, E, H)
  combine = pl.pallas_call(
      functools.partial(_tc_combine_body, N),
      out_shape=(nh, nh), compiler_params=vmem100,
  )
  combine_last = pl.pallas_call(
      functools.partial(_tc_combine_last_body, N),
      out_shape=nh, compiler_params=vmem100,
  )
  pool_call = pl.pallas_call(
      functools.partial(_tc_pool_body, N, H, G),
      out_shape=jax.ShapeDtypeStruct((G, 2 * H), jnp.float32),
      compiler_params=vmem100,
  )

  pools = []
  h_prev = None
  for bc, wcn in ((bc0, Wc1), (bc1, Wc2), (bc2, None)):
    accp = msg(g, src, dst, ew)
    if h_prev is not None:
      pools.append(pool_call(h_prev, bat2))
    if wcn is None:
      h_prev = combine_last(accp, g, dinv2, bc.reshape(1, H))
    else:
      h_prev, g = combine(accp, g, dinv2, bc.reshape(1, H), wcn)
  pools.append(pool_call(h_prev, bat2))

  return jnp.concatenate(pools, axis=1)
